# baseline (speedup 1.0000x reference)
import functools

import jax
import jax.numpy as jnp
from jax import lax
from jax.experimental import pallas as pl
from jax.experimental.pallas import tpu as pltpu
from jax.experimental.pallas import tpu_sc as plsc

B, L, D = 16384, 200, 128
NC, NS = 2, 16
NW = NC * NS
RPW = B // NW
XBLK = 64
ROW_W = D * L
PAIRS_PER_XBLK = XBLK // 2

CH_STARTS = tuple(list(range(0, 192, 16)) + [L - 16])


def _compute_row(xbuf, xoff, w0v, dwv, obuf, obase):
    def dbody(dd, _):
        w0 = w0v[pl.ds(dd * 16, 16)]
        dw = dwv[pl.ds(dd * 16, 16)]
        ob = obase + dd * L
        for off in CH_STARTS:
            xf = xbuf[pl.ds(xoff + off, 16)].astype(jnp.float32)
            obuf[pl.ds(ob + off, 16)] = w0 + dw * xf
        return 0

    lax.fori_loop(0, D, dbody, 0)


def _sc_body(x_hbm, w0_hbm, dw_hbm, out_hbm, xbuf, obuf, w0v, dwv, sem0, sem1):
    wid = lax.axis_index("s") * NC + lax.axis_index("c")
    base_row = wid * RPW
    pltpu.sync_copy(w0_hbm, w0v)
    pltpu.sync_copy(dw_hbm, dwv)

    def pair_body(p, _):
        row0 = base_row + 2 * p

        @pl.when(p % PAIRS_PER_XBLK == 0)
        def _stage_x():
            blk_row = base_row + (p // PAIRS_PER_XBLK) * XBLK
            pltpu.sync_copy(x_hbm.at[pl.ds(blk_row * L, XBLK * L)], xbuf)

        xoff0 = (p % PAIRS_PER_XBLK) * 2 * L

        @pl.when(p >= 1)
        def _wait0():
            pltpu.make_async_copy(
                obuf.at[pl.ds(0, ROW_W)],
                out_hbm.at[pl.ds((row0 - 2) * ROW_W, ROW_W)], sem0).wait()

        _compute_row(xbuf, xoff0, w0v, dwv, obuf, 0)
        pltpu.make_async_copy(
            obuf.at[pl.ds(0, ROW_W)],
            out_hbm.at[pl.ds(row0 * ROW_W, ROW_W)], sem0).start()

        @pl.when(p >= 1)
        def _wait1():
            pltpu.make_async_copy(
                obuf.at[pl.ds(ROW_W, ROW_W)],
                out_hbm.at[pl.ds((row0 - 1) * ROW_W, ROW_W)], sem1).wait()

        _compute_row(xbuf, xoff0 + L, w0v, dwv, obuf, ROW_W)
        pltpu.make_async_copy(
            obuf.at[pl.ds(ROW_W, ROW_W)],
            out_hbm.at[pl.ds((row0 + 1) * ROW_W, ROW_W)], sem1).start()
        return 0

    lax.fori_loop(0, RPW // 2, pair_body, 0)
    last = base_row + RPW
    pltpu.make_async_copy(
        obuf.at[pl.ds(0, ROW_W)],
        out_hbm.at[pl.ds((last - 2) * ROW_W, ROW_W)], sem0).wait()
    pltpu.make_async_copy(
        obuf.at[pl.ds(ROW_W, ROW_W)],
        out_hbm.at[pl.ds((last - 1) * ROW_W, ROW_W)], sem1).wait()


_sc_call = functools.partial(
    pl.kernel,
    out_type=jax.ShapeDtypeStruct((B * ROW_W,), jnp.float32),
    mesh=plsc.VectorSubcoreMesh(core_axis_name="c", subcore_axis_name="s"),
    scratch_types=[
        pltpu.VMEM((XBLK * L,), jnp.int32),
        pltpu.VMEM((2 * ROW_W,), jnp.float32),
        pltpu.VMEM((D * 16,), jnp.float32),
        pltpu.VMEM((D * 16,), jnp.float32),
        pltpu.SemaphoreType.DMA,
        pltpu.SemaphoreType.DMA,
    ],
)(_sc_body)


def kernel(x, embedding_weight):
    x = x.astype(jnp.int32).reshape(-1)
    w0 = embedding_weight[0]
    dw = embedding_weight[1] - embedding_weight[0]
    w0rep = jnp.repeat(w0, 16)
    dwrep = jnp.repeat(dw, 16)
    out = _sc_call(x, w0rep, dwrep)
    return out.reshape(B, D, L)


# speedup vs baseline: 2.0444x; 2.0444x over previous
import functools

import jax
import jax.numpy as jnp
from jax import lax
from jax.experimental import pallas as pl
from jax.experimental.pallas import tpu as pltpu
from jax.experimental.pallas import tpu_sc as plsc

B, L, D = 16384, 200, 128
NC, NS = 2, 16
NW = NC * NS
RPW = B // NW
XBLK = 64
ROW_W = D * L
PAIRS_PER_XBLK = XBLK // 2

CH_STARTS = tuple(list(range(0, 192, 16)) + [L - 16])


def _compute_row(xbuf, xoff, w0v, dwv, obuf, obase):
    xfs = tuple(
        xbuf[pl.ds(xoff + off, 16)].astype(jnp.float32) for off in CH_STARTS
    )

    @plsc.parallel_loop(0, D, step=1, unroll=4, carry=xfs)
    def dbody(dd, xfs_c):
        w0 = w0v[pl.ds(dd * 16, 16)]
        dw = dwv[pl.ds(dd * 16, 16)]
        ob = obase + dd * L
        for off, xf in zip(CH_STARTS, xfs_c):
            obuf[pl.ds(ob + off, 16)] = w0 + dw * xf
        return xfs_c


def _sc_body(x_hbm, w0_hbm, dw_hbm, out_hbm, xbuf, obuf, w0v, dwv, sem0, sem1):
    wid = lax.axis_index("s") * NC + lax.axis_index("c")
    base_row = wid * RPW
    pltpu.sync_copy(w0_hbm, w0v)
    pltpu.sync_copy(dw_hbm, dwv)

    def pair_body(p, _):
        row0 = base_row + 2 * p

        @pl.when(p % PAIRS_PER_XBLK == 0)
        def _stage_x():
            blk_row = base_row + (p // PAIRS_PER_XBLK) * XBLK
            pltpu.sync_copy(x_hbm.at[pl.ds(blk_row * L, XBLK * L)], xbuf)

        xoff0 = (p % PAIRS_PER_XBLK) * 2 * L

        @pl.when(p >= 1)
        def _wait0():
            pltpu.make_async_copy(
                obuf.at[pl.ds(0, ROW_W)],
                out_hbm.at[pl.ds((row0 - 2) * ROW_W, ROW_W)], sem0).wait()

        _compute_row(xbuf, xoff0, w0v, dwv, obuf, 0)
        pltpu.make_async_copy(
            obuf.at[pl.ds(0, ROW_W)],
            out_hbm.at[pl.ds(row0 * ROW_W, ROW_W)], sem0).start()

        @pl.when(p >= 1)
        def _wait1():
            pltpu.make_async_copy(
                obuf.at[pl.ds(ROW_W, ROW_W)],
                out_hbm.at[pl.ds((row0 - 1) * ROW_W, ROW_W)], sem1).wait()

        _compute_row(xbuf, xoff0 + L, w0v, dwv, obuf, ROW_W)
        pltpu.make_async_copy(
            obuf.at[pl.ds(ROW_W, ROW_W)],
            out_hbm.at[pl.ds((row0 + 1) * ROW_W, ROW_W)], sem1).start()
        return 0

    lax.fori_loop(0, RPW // 2, pair_body, 0)
    last = base_row + RPW
    pltpu.make_async_copy(
        obuf.at[pl.ds(0, ROW_W)],
        out_hbm.at[pl.ds((last - 2) * ROW_W, ROW_W)], sem0).wait()
    pltpu.make_async_copy(
        obuf.at[pl.ds(ROW_W, ROW_W)],
        out_hbm.at[pl.ds((last - 1) * ROW_W, ROW_W)], sem1).wait()


_sc_call = functools.partial(
    pl.kernel,
    out_type=jax.ShapeDtypeStruct((B * ROW_W,), jnp.float32),
    mesh=plsc.VectorSubcoreMesh(core_axis_name="c", subcore_axis_name="s"),
    scratch_types=[
        pltpu.VMEM((XBLK * L,), jnp.int32),
        pltpu.VMEM((2 * ROW_W,), jnp.float32),
        pltpu.VMEM((D * 16,), jnp.float32),
        pltpu.VMEM((D * 16,), jnp.float32),
        pltpu.SemaphoreType.DMA,
        pltpu.SemaphoreType.DMA,
    ],
)(_sc_body)


def kernel(x, embedding_weight):
    x = x.astype(jnp.int32).reshape(-1)
    w0 = embedding_weight[0]
    dw = embedding_weight[1] - embedding_weight[0]
    w0rep = jnp.repeat(w0, 16)
    dwrep = jnp.repeat(dw, 16)
    out = _sc_call(x, w0rep, dwrep)
    return out.reshape(B, D, L)


# speedup vs baseline: 8.5536x; 4.1839x over previous
import functools

import jax
import jax.numpy as jnp
from jax import lax
from jax.experimental import pallas as pl
from jax.experimental.pallas import tpu as pltpu
from jax.experimental.pallas import tpu_sc as plsc

B, L, D = 16384, 200, 128
NC, NS = 2, 16
NW = NC * NS
RPW = B // NW
XBLK = 64
ROW_W = D * L
PAIRS_PER_XBLK = XBLK // 2
NDCH = D // 16


N_LG = 13


def _compute_row(xbuf, xoff, w0v, dwv, obuf, obase):
    w0s = tuple(w0v[pl.ds(k * 16, 16)] for k in range(NDCH))
    dws = tuple(dwv[pl.ds(k * 16, 16)] for k in range(NDCH))

    @plsc.parallel_loop(0, N_LG, step=1, unroll=2, carry=(w0s, dws))
    def gbody(lg, c):
        w0c, dwc = c
        lstart = jnp.minimum(lg * 16, L - 16)
        xc = xbuf[pl.ds(xoff + lstart, 16)].astype(jnp.float32)
        ob = obase + lstart * D
        for j in range(16):
            xf = jnp.full((16,), xc[j], jnp.float32)
            o = ob + j * D
            for k in range(NDCH):
                obuf[pl.ds(o + k * 16, 16)] = w0c[k] + dwc[k] * xf
        return c


def _sc_body(x_hbm, w0_hbm, dw_hbm, out_hbm, xbuf, obuf, w0v, dwv, sem0, sem1):
    wid = lax.axis_index("s") * NC + lax.axis_index("c")
    base_row = wid * RPW
    pltpu.sync_copy(w0_hbm, w0v)
    pltpu.sync_copy(dw_hbm, dwv)

    def pair_body(p, _):
        row0 = base_row + 2 * p

        @pl.when(p % PAIRS_PER_XBLK == 0)
        def _stage_x():
            blk_row = base_row + (p // PAIRS_PER_XBLK) * XBLK
            pltpu.sync_copy(x_hbm.at[pl.ds(blk_row * L, XBLK * L)], xbuf)

        xoff0 = (p % PAIRS_PER_XBLK) * 2 * L

        @pl.when(p >= 1)
        def _wait0():
            pltpu.make_async_copy(
                obuf.at[pl.ds(0, ROW_W)],
                out_hbm.at[pl.ds((row0 - 2) * ROW_W, ROW_W)], sem0).wait()

        _compute_row(xbuf, xoff0, w0v, dwv, obuf, 0)
        pltpu.make_async_copy(
            obuf.at[pl.ds(0, ROW_W)],
            out_hbm.at[pl.ds(row0 * ROW_W, ROW_W)], sem0).start()

        @pl.when(p >= 1)
        def _wait1():
            pltpu.make_async_copy(
                obuf.at[pl.ds(ROW_W, ROW_W)],
                out_hbm.at[pl.ds((row0 - 1) * ROW_W, ROW_W)], sem1).wait()

        _compute_row(xbuf, xoff0 + L, w0v, dwv, obuf, ROW_W)
        pltpu.make_async_copy(
            obuf.at[pl.ds(ROW_W, ROW_W)],
            out_hbm.at[pl.ds((row0 + 1) * ROW_W, ROW_W)], sem1).start()
        return 0

    lax.fori_loop(0, RPW // 2, pair_body, 0)
    last = base_row + RPW
    pltpu.make_async_copy(
        obuf.at[pl.ds(0, ROW_W)],
        out_hbm.at[pl.ds((last - 2) * ROW_W, ROW_W)], sem0).wait()
    pltpu.make_async_copy(
        obuf.at[pl.ds(ROW_W, ROW_W)],
        out_hbm.at[pl.ds((last - 1) * ROW_W, ROW_W)], sem1).wait()


_sc_call = functools.partial(
    pl.kernel,
    out_type=jax.ShapeDtypeStruct((B * ROW_W,), jnp.float32),
    mesh=plsc.VectorSubcoreMesh(core_axis_name="c", subcore_axis_name="s"),
    scratch_types=[
        pltpu.VMEM((XBLK * L,), jnp.int32),
        pltpu.VMEM((2 * ROW_W,), jnp.float32),
        pltpu.VMEM((D,), jnp.float32),
        pltpu.VMEM((D,), jnp.float32),
        pltpu.SemaphoreType.DMA,
        pltpu.SemaphoreType.DMA,
    ],
)(_sc_body)


def kernel(x, embedding_weight):
    x = x.astype(jnp.int32).reshape(-1)
    w0 = embedding_weight[0]
    dw = embedding_weight[1] - embedding_weight[0]
    out = _sc_call(x, w0, dw)
    return jnp.transpose(out.reshape(B, L, D), (0, 2, 1))


# speedup vs baseline: 17.7307x; 2.0729x over previous
import functools

import jax
import jax.numpy as jnp
from jax import lax
from jax.experimental import pallas as pl
from jax.experimental.pallas import tpu as pltpu
from jax.experimental.pallas import tpu_sc as plsc

B, L, D = 16384, 200, 128
NC, NS = 2, 16
NW = NC * NS
RPW = B // NW
XBLK = 64
ROW_W = D * L
PAIRS_PER_XBLK = XBLK // 2
NDCH = D // 16


N_LG = 13


def _compute_row(xbuf, xoff, w0v, dwv, obuf, obase):
    w0s = tuple(w0v[pl.ds(k * 16, 16)] for k in range(NDCH))
    dws = tuple(dwv[pl.ds(k * 16, 16)] for k in range(NDCH))

    @plsc.parallel_loop(0, N_LG, step=1, unroll=4, carry=(w0s, dws))
    def gbody(lg, c):
        w0c, dwc = c
        lstart = jnp.minimum(lg * 16, L - 16)
        xc = xbuf[pl.ds(xoff + lstart, 16)].astype(jnp.float32)
        ob = obase + lstart * D
        for j in range(16):
            xf = jnp.full((16,), xc[j], jnp.float32)
            o = ob + j * D
            for k in range(NDCH):
                obuf[pl.ds(o + k * 16, 16)] = w0c[k] + dwc[k] * xf
        return c


def _sc_body(x_hbm, w0_hbm, dw_hbm, out_hbm, xbuf, obuf, w0v, dwv, sem0, sem1):
    wid = lax.axis_index("s") * NC + lax.axis_index("c")
    base_row = wid * RPW
    pltpu.sync_copy(w0_hbm, w0v)
    pltpu.sync_copy(dw_hbm, dwv)

    def pair_body(p, _):
        row0 = base_row + 2 * p

        @pl.when(p % PAIRS_PER_XBLK == 0)
        def _stage_x():
            blk_row = base_row + (p // PAIRS_PER_XBLK) * XBLK
            pltpu.sync_copy(x_hbm.at[pl.ds(blk_row * L, XBLK * L)], xbuf)

        xoff0 = (p % PAIRS_PER_XBLK) * 2 * L

        @pl.when(p >= 1)
        def _wait0():
            pltpu.make_async_copy(
                obuf.at[pl.ds(0, ROW_W)],
                out_hbm.at[pl.ds((row0 - 2) * ROW_W, ROW_W)], sem0).wait()

        _compute_row(xbuf, xoff0, w0v, dwv, obuf, 0)
        pltpu.make_async_copy(
            obuf.at[pl.ds(0, ROW_W)],
            out_hbm.at[pl.ds(row0 * ROW_W, ROW_W)], sem0).start()

        @pl.when(p >= 1)
        def _wait1():
            pltpu.make_async_copy(
                obuf.at[pl.ds(ROW_W, ROW_W)],
                out_hbm.at[pl.ds((row0 - 1) * ROW_W, ROW_W)], sem1).wait()

        _compute_row(xbuf, xoff0 + L, w0v, dwv, obuf, ROW_W)
        pltpu.make_async_copy(
            obuf.at[pl.ds(ROW_W, ROW_W)],
            out_hbm.at[pl.ds((row0 + 1) * ROW_W, ROW_W)], sem1).start()
        return 0

    lax.fori_loop(0, RPW // 2, pair_body, 0)
    last = base_row + RPW
    pltpu.make_async_copy(
        obuf.at[pl.ds(0, ROW_W)],
        out_hbm.at[pl.ds((last - 2) * ROW_W, ROW_W)], sem0).wait()
    pltpu.make_async_copy(
        obuf.at[pl.ds(ROW_W, ROW_W)],
        out_hbm.at[pl.ds((last - 1) * ROW_W, ROW_W)], sem1).wait()


_sc_call = functools.partial(
    pl.kernel,
    out_type=jax.ShapeDtypeStruct((B * ROW_W,), jnp.float32),
    mesh=plsc.VectorSubcoreMesh(core_axis_name="c", subcore_axis_name="s"),
    scratch_types=[
        pltpu.VMEM((XBLK * L,), jnp.int32),
        pltpu.VMEM((2 * ROW_W,), jnp.float32),
        pltpu.VMEM((D,), jnp.float32),
        pltpu.VMEM((D,), jnp.float32),
        pltpu.SemaphoreType.DMA,
        pltpu.SemaphoreType.DMA,
    ],
)(_sc_body)


def kernel(x, embedding_weight):
    x = x.astype(jnp.int32).reshape(-1)
    w0 = embedding_weight[0]
    dw = embedding_weight[1] - embedding_weight[0]
    out = _sc_call(x, w0, dw)
    return jnp.transpose(out.reshape(B, L, D), (0, 2, 1))
